# baseline (speedup 1.0000x reference)
import functools

import jax
import jax.numpy as jnp
from jax import lax
from jax.experimental import pallas as pl
from jax.experimental.pallas import tpu as pltpu
from jax.experimental.pallas import tpu_sc as plsc

N_CLS = 1000
N_CTX = 16
D = 512
SEQ = 77
SUF = SEQ - 1 - N_CTX
SUF_MAIN = 56
SUF_TAIL = 8
SUF_PAD = SUF_MAIN + SUF_TAIL
B = 1024
SEQ_PAD = 128

NC, NS = 2, 16
NW = NC * NS
NCHUNK = 4
BC = B // NCHUNK
BPW = BC // NW
CTX_CH = 2
TAIL_CH = 4


def _sc_body(cls_w, cls_p, cls_c, cls_t,
             pre_n, ctx_n, suf_n, pre_p, ctx_p, suf_p,
             tail_n, tail_p, tok_n, tok_p,
             out_pre, out_ctx, out_suf, out_tok,
             idx_v, idxp_v, idxc_v, idxt_v,
             pbuf, tbuf, cbuf0, cbuf1, sbuf0, sbuf1, lbuf,
             gsem_p, ssem_p, gsem_t, ssem_t, gsem_l, ssem_l,
             gsem_c0, gsem_c1, ssem_c0, ssem_c1,
             gsem_s0, gsem_s1, ssem_s0, ssem_s1):
    wid = lax.axis_index("s") * NC + lax.axis_index("c")
    base = wid * BPW

    pltpu.sync_copy(cls_w.at[wid], idx_v)
    pltpu.sync_copy(cls_p.at[wid], idxp_v)
    pltpu.sync_copy(cls_c.at[wid], idxc_v)
    pltpu.sync_copy(cls_t.at[wid], idxt_v)

    slots = {
        "s0": (sbuf0, gsem_s0, ssem_s0),
        "s1": (sbuf1, gsem_s1, ssem_s1),
        "c0": (cbuf0, gsem_c0, ssem_c0),
        "c1": (cbuf1, gsem_c1, ssem_c1),
        "p": (pbuf, gsem_p, ssem_p),
        "t": (tbuf, gsem_t, ssem_t),
        "l": (lbuf, gsem_l, ssem_l),
    }

    task_list = []
    halves = ((pre_n, ctx_n, suf_n, tail_n, tok_n, 0),
              (pre_p, ctx_p, suf_p, tail_p, tok_p, 1))
    for pol, (pre_t, ctx_t, suf_t, tail_t, tok_t, _) in enumerate(halves):
        ob = pol * BC + base
        for j in range(BPW):
            idx = idxc_v.at[pol * (BPW // CTX_CH) + j // CTX_CH,
                            pl.ds(j % CTX_CH, 1)]
            dst = out_suf.at[pl.ds(ob + j, 1), pl.ds(0, SUF_MAIN), :]
            task_list.append((suf_t.at[idx, pl.ds(0, SUF_MAIN), :], dst,
                              "s0" if j % 2 == 0 else "s1"))
            if j % CTX_CH == 0:
                jc = j // CTX_CH
                idxc = idxc_v.at[pol * (BPW // CTX_CH) + jc]
                cdst = out_ctx.at[pl.ds(ob + jc * CTX_CH, CTX_CH)]
                task_list.append((ctx_t.at[idxc], cdst,
                                  "c0" if jc % 2 == 0 else "c1"))
            if j % TAIL_CH == 0:
                jt = j // TAIL_CH
                idxt = idxt_v.at[pol * (BPW // TAIL_CH) + jt]
                ldst = out_suf.at[pl.ds(ob + jt * TAIL_CH, TAIL_CH),
                                  pl.ds(SUF_MAIN, SUF_TAIL), :]
                task_list.append((tail_t.at[idxt], ldst, "l"))
            if j == 1:
                task_list.append((pre_t.at[idxp_v.at[pol]],
                                  out_pre.at[pl.ds(ob, BPW)], "p"))
            if j == 3:
                task_list.append((tok_t.at[idxp_v.at[pol]],
                                  out_tok.at[pl.ds(ob, BPW), :], "t"))

    last_scatter = {}
    prev = None
    for src, dst, slot in task_list:
        buf, gsem, _ = slots[slot]
        if slot in last_scatter:
            last_scatter.pop(slot).wait()
        g = pltpu.async_copy(src, buf, gsem)
        if prev is not None:
            pg, pdst, pslot = prev
            pbuf_, _, pssem = slots[pslot]
            pg.wait()
            last_scatter[pslot] = pltpu.async_copy(pbuf_, pdst, pssem)
        prev = (g, dst, slot)
    pg, pdst, pslot = prev
    pbuf_, _, pssem = slots[pslot]
    pg.wait()
    last_scatter[pslot] = pltpu.async_copy(pbuf_, pdst, pssem)
    for s in last_scatter.values():
        s.wait()


@functools.lru_cache(maxsize=None)
def _make_call():
    mesh = plsc.VectorSubcoreMesh(core_axis_name="c", subcore_axis_name="s",
                                  num_cores=NC, num_subcores=NS)
    return pl.kernel(
        _sc_body,
        out_type=(
            jax.ShapeDtypeStruct((2 * BC, D), jnp.float32),
            jax.ShapeDtypeStruct((2 * BC, N_CTX, D), jnp.float32),
            jax.ShapeDtypeStruct((2 * BC, SUF_PAD, D), jnp.float32),
            jax.ShapeDtypeStruct((2 * BC, SEQ_PAD), jnp.int32),
        ),
        mesh=mesh,
        scratch_types=[
            pltpu.VMEM((BPW,), jnp.int32),
            pltpu.VMEM((2, BPW), jnp.int32),
            pltpu.VMEM((2 * BPW // CTX_CH, CTX_CH), jnp.int32),
            pltpu.VMEM((2 * BPW // TAIL_CH, TAIL_CH), jnp.int32),
            pltpu.VMEM((BPW, D), jnp.float32),
            pltpu.VMEM((BPW, SEQ_PAD), jnp.int32),
            pltpu.VMEM((CTX_CH, N_CTX, D), jnp.float32),
            pltpu.VMEM((CTX_CH, N_CTX, D), jnp.float32),
            pltpu.VMEM((1, SUF_MAIN, D), jnp.float32),
            pltpu.VMEM((1, SUF_MAIN, D), jnp.float32),
            pltpu.VMEM((TAIL_CH, SUF_TAIL, D), jnp.float32),
        ] + [pltpu.SemaphoreType.DMA] * 14,
    )


@jax.jit
def _prompt_gather(cls_id, ctx_pos, ctx_neg, pre_pos2, suf_pos,
                   pre_neg2, suf_neg, tail_pos, tail_neg, tok_neg, tok_pos):
    call = _make_call()
    prompts = jnp.zeros((2 * B, SEQ, D), jnp.float32)
    tokenized = jnp.zeros((2 * B, SEQ_PAD), jnp.int32)
    for i in range(NCHUNK):
        cls_c = lax.dynamic_slice_in_dim(cls_id, i * BC, BC)
        cls_w = cls_c.reshape(NW, BPW)
        cls2 = jnp.concatenate([cls_w] * 2, axis=1)
        pre, ctx, suf, tok = call(
            cls_w,
            cls2.reshape(NW, 2, BPW),
            cls2.reshape(NW, 2 * BPW // CTX_CH, CTX_CH),
            cls2.reshape(NW, 2 * BPW // TAIL_CH, TAIL_CH),
            pre_neg2, ctx_neg, suf_neg,
            pre_pos2, ctx_pos, suf_pos,
            tail_neg, tail_pos, tok_neg, tok_pos)
        asm = jnp.concatenate(
            [pre.reshape(2 * BC, 1, D), ctx, suf[:, :SUF, :]], axis=1)
        prompts = lax.dynamic_update_slice(prompts, asm[:BC], (i * BC, 0, 0))
        prompts = lax.dynamic_update_slice(prompts, asm[BC:],
                                           (B + i * BC, 0, 0))
        tokenized = lax.dynamic_update_slice(tokenized, tok[:BC],
                                             (i * BC, 0))
        tokenized = lax.dynamic_update_slice(tokenized, tok[BC:],
                                             (B + i * BC, 0))
    return prompts, tokenized[:, :SEQ]


def kernel(cls_id, ctx_pos, ctx_neg, token_prefix_pos, token_suffix_pos,
           token_prefix_neg, token_suffix_neg, tokenized_prompts):
    n_cls = ctx_pos.shape[0]
    pad_tail = ((0, 0), (0, SUF_TAIL - (SUF - SUF_MAIN)), (0, 0))
    return _prompt_gather(
        cls_id, ctx_pos, ctx_neg,
        token_prefix_pos.reshape(n_cls, D),
        token_suffix_pos,
        token_prefix_neg.reshape(n_cls, D),
        token_suffix_neg,
        jnp.pad(token_suffix_pos[:, SUF_MAIN:, :], pad_tail),
        jnp.pad(token_suffix_neg[:, SUF_MAIN:, :], pad_tail),
        jnp.pad(tokenized_prompts[:n_cls], ((0, 0), (0, SEQ_PAD - SEQ))),
        jnp.pad(tokenized_prompts[n_cls:], ((0, 0), (0, SEQ_PAD - SEQ))),
    )


# speedup vs baseline: 1.3529x; 1.3529x over previous
import functools

import jax
import jax.numpy as jnp
from jax import lax
from jax.experimental import pallas as pl
from jax.experimental.pallas import tpu as pltpu
from jax.experimental.pallas import tpu_sc as plsc

N_CLS = 1000
N_CTX = 16
D = 512
SEQ = 77
SUF = SEQ - 1 - N_CTX
SUF_MAIN = 56
SUF_TAIL = 8
SUF_PAD = SUF_MAIN + SUF_TAIL
B = 1024
SEQ_PAD = 128

NC, NS = 2, 16
NW = NC * NS
NCHUNK = 4
BC = B // NCHUNK
BPW = BC // NW
CTX_CH = 2
TAIL_CH = 4


def _sc_body(cls_w, cls_p, cls_c, cls_t,
             pre_n, ctx_n, suf_n, pre_p, ctx_p, suf_p,
             tail_n, tail_p, tok_n, tok_p,
             out_pre, out_ctx, out_suf, out_tok,
             idx_v, idxp_v, idxc_v, idxt_v,
             pbuf, tbuf, cbuf0, cbuf1, sbuf0, sbuf1, lbuf,
             gsem_p, ssem_p, gsem_t, ssem_t, gsem_l, ssem_l,
             gsem_c0, gsem_c1, ssem_c0, ssem_c1,
             gsem_s0, gsem_s1, ssem_s0, ssem_s1):
    wid = lax.axis_index("s") * NC + lax.axis_index("c")
    base = wid * BPW

    pltpu.sync_copy(cls_w.at[wid], idx_v)
    pltpu.sync_copy(cls_p.at[wid], idxp_v)
    pltpu.sync_copy(cls_c.at[wid], idxc_v)
    pltpu.sync_copy(cls_t.at[wid], idxt_v)

    slots = {
        "s0": (sbuf0, gsem_s0, ssem_s0),
        "s1": (sbuf1, gsem_s1, ssem_s1),
        "c0": (cbuf0, gsem_c0, ssem_c0),
        "c1": (cbuf1, gsem_c1, ssem_c1),
        "p": (pbuf, gsem_p, ssem_p),
        "t": (tbuf, gsem_t, ssem_t),
        "l": (lbuf, gsem_l, ssem_l),
    }

    task_list = []
    halves = ((pre_n, ctx_n, suf_n, tail_n, tok_n, 0),
              (pre_p, ctx_p, suf_p, tail_p, tok_p, 1))
    for pol, (pre_t, ctx_t, suf_t, tail_t, tok_t, _) in enumerate(halves):
        ob = pol * BC + base
        for j in range(BPW):
            idx = idxc_v.at[pol * (BPW // CTX_CH) + j // CTX_CH,
                            pl.ds(j % CTX_CH, 1)]
            dst = out_suf.at[pl.ds(ob + j, 1), pl.ds(0, SUF_MAIN), :]
            task_list.append((suf_t.at[idx, pl.ds(0, SUF_MAIN), :], dst,
                              "s0" if j % 2 == 0 else "s1"))
            if j % CTX_CH == 0:
                jc = j // CTX_CH
                idxc = idxc_v.at[pol * (BPW // CTX_CH) + jc]
                cdst = out_ctx.at[pl.ds(ob + jc * CTX_CH, CTX_CH)]
                task_list.append((ctx_t.at[idxc], cdst,
                                  "c0" if jc % 2 == 0 else "c1"))
            if j % TAIL_CH == 0:
                jt = j // TAIL_CH
                idxt = idxt_v.at[pol * (BPW // TAIL_CH) + jt]
                ldst = out_suf.at[pl.ds(ob + jt * TAIL_CH, TAIL_CH),
                                  pl.ds(SUF_MAIN, SUF_TAIL), :]
                task_list.append((tail_t.at[idxt], ldst, "l"))
            if j == 1:
                task_list.append((pre_t.at[idxp_v.at[pol]],
                                  out_pre.at[pl.ds(ob, BPW)], "p"))
            if j == 3:
                task_list.append((tok_t.at[idxp_v.at[pol]],
                                  out_tok.at[pl.ds(ob, BPW), :], "t"))

    last_scatter = {}
    prev = None
    for src, dst, slot in task_list:
        buf, gsem, _ = slots[slot]
        if slot in last_scatter:
            last_scatter.pop(slot).wait()
        g = pltpu.async_copy(src, buf, gsem)
        if prev is not None:
            pg, pdst, pslot = prev
            pbuf_, _, pssem = slots[pslot]
            pg.wait()
            last_scatter[pslot] = pltpu.async_copy(pbuf_, pdst, pssem)
        prev = (g, dst, slot)
    pg, pdst, pslot = prev
    pbuf_, _, pssem = slots[pslot]
    pg.wait()
    last_scatter[pslot] = pltpu.async_copy(pbuf_, pdst, pssem)
    for s in last_scatter.values():
        s.wait()


@functools.lru_cache(maxsize=None)
def _make_call():
    mesh = plsc.VectorSubcoreMesh(core_axis_name="c", subcore_axis_name="s",
                                  num_cores=NC, num_subcores=NS)
    return pl.kernel(
        _sc_body,
        out_type=(
            jax.ShapeDtypeStruct((2 * BC, D), jnp.float32),
            jax.ShapeDtypeStruct((2 * BC, N_CTX, D), jnp.float32),
            jax.ShapeDtypeStruct((2 * BC, SUF_PAD, D), jnp.float32),
            jax.ShapeDtypeStruct((2 * BC, SEQ_PAD), jnp.int32),
        ),
        mesh=mesh,
        scratch_types=[
            pltpu.VMEM((BPW,), jnp.int32),
            pltpu.VMEM((2, BPW), jnp.int32),
            pltpu.VMEM((2 * BPW // CTX_CH, CTX_CH), jnp.int32),
            pltpu.VMEM((2 * BPW // TAIL_CH, TAIL_CH), jnp.int32),
            pltpu.VMEM((BPW, D), jnp.float32),
            pltpu.VMEM((BPW, SEQ_PAD), jnp.int32),
            pltpu.VMEM((CTX_CH, N_CTX, D), jnp.float32),
            pltpu.VMEM((CTX_CH, N_CTX, D), jnp.float32),
            pltpu.VMEM((1, SUF_MAIN, D), jnp.float32),
            pltpu.VMEM((1, SUF_MAIN, D), jnp.float32),
            pltpu.VMEM((TAIL_CH, SUF_TAIL, D), jnp.float32),
        ] + [pltpu.SemaphoreType.DMA] * 14,
    )


def _tc_asm_body(*refs):
    pre_ref, ctx_ref, suf_ref, tokc_ref = refs[-6:-2]
    out_ref, otok_ref = refs[-2:]
    out_ref[:, 0, :] = pre_ref[...]
    out_ref[:, 1:1 + N_CTX, :] = ctx_ref[...]
    out_ref[:, 1 + N_CTX:SEQ, :] = suf_ref[:, :SUF, :]
    otok_ref[...] = tokc_ref[...]


def _tc_asm(i, prompts, tokenized, pre, ctx, suf, tok):
    grid = (2, BC // 8)
    nb, cb = B // 8, BC // 8

    def seg_map(pol, s):
        return (pol * cb + s, 0)

    def seg3_map(pol, s):
        return (pol * cb + s, 0, 0)

    def out_map(pol, s):
        return (pol * nb + i * cb + s, 0, 0)

    def otok_map(pol, s):
        return (pol * nb + i * cb + s, 0)

    seg_specs = [
        pl.BlockSpec((8, D), seg_map),
        pl.BlockSpec((8, N_CTX, D), seg3_map),
        pl.BlockSpec((8, SUF_PAD, D), seg3_map),
        pl.BlockSpec((8, SEQ_PAD), seg_map),
    ]
    out_shape = (
        jax.ShapeDtypeStruct((2 * B, SEQ, D), jnp.float32),
        jax.ShapeDtypeStruct((2 * B, SEQ_PAD), jnp.int32),
    )
    out_specs = (
        pl.BlockSpec((8, SEQ, D), out_map),
        pl.BlockSpec((8, SEQ_PAD), otok_map),
    )
    if i == 0:
        return pl.pallas_call(
            _tc_asm_body, grid=grid, in_specs=seg_specs,
            out_specs=out_specs, out_shape=out_shape,
        )(pre, ctx, suf, tok)
    return pl.pallas_call(
        _tc_asm_body, grid=grid,
        in_specs=[pl.BlockSpec(memory_space=pl.ANY),
                  pl.BlockSpec(memory_space=pl.ANY)] + seg_specs,
        out_specs=out_specs, out_shape=out_shape,
        input_output_aliases={0: 0, 1: 1},
    )(prompts, tokenized, pre, ctx, suf, tok)


@jax.jit
def _prompt_gather(cls_id, ctx_pos, ctx_neg, pre_pos2, suf_pos,
                   pre_neg2, suf_neg, tail_pos, tail_neg, tok_neg, tok_pos):
    call = _make_call()
    prompts, tokenized = None, None
    for i in range(NCHUNK):
        cls_c = lax.dynamic_slice_in_dim(cls_id, i * BC, BC)
        cls_w = cls_c.reshape(NW, BPW)
        cls2 = jnp.concatenate([cls_w] * 2, axis=1)
        pre, ctx, suf, tok = call(
            cls_w,
            cls2.reshape(NW, 2, BPW),
            cls2.reshape(NW, 2 * BPW // CTX_CH, CTX_CH),
            cls2.reshape(NW, 2 * BPW // TAIL_CH, TAIL_CH),
            pre_neg2, ctx_neg, suf_neg,
            pre_pos2, ctx_pos, suf_pos,
            tail_neg, tail_pos, tok_neg, tok_pos)
        prompts, tokenized = _tc_asm(i, prompts, tokenized,
                                     pre, ctx, suf, tok)
    return prompts, tokenized[:, :SEQ]


def kernel(cls_id, ctx_pos, ctx_neg, token_prefix_pos, token_suffix_pos,
           token_prefix_neg, token_suffix_neg, tokenized_prompts):
    n_cls = ctx_pos.shape[0]
    pad_tail = ((0, 0), (0, SUF_TAIL - (SUF - SUF_MAIN)), (0, 0))
    return _prompt_gather(
        cls_id, ctx_pos, ctx_neg,
        token_prefix_pos.reshape(n_cls, D),
        token_suffix_pos,
        token_prefix_neg.reshape(n_cls, D),
        token_suffix_neg,
        jnp.pad(token_suffix_pos[:, SUF_MAIN:, :], pad_tail),
        jnp.pad(token_suffix_neg[:, SUF_MAIN:, :], pad_tail),
        jnp.pad(tokenized_prompts[:n_cls], ((0, 0), (0, SEQ_PAD - SEQ))),
        jnp.pad(tokenized_prompts[n_cls:], ((0, 0), (0, SEQ_PAD - SEQ))),
    )


# speedup vs baseline: 1.3982x; 1.0335x over previous
import functools

import jax
import jax.numpy as jnp
from jax import lax
from jax.experimental import pallas as pl
from jax.experimental.pallas import tpu as pltpu
from jax.experimental.pallas import tpu_sc as plsc

N_CLS = 1000
N_CTX = 16
D = 512
SEQ = 77
SUF = SEQ - 1 - N_CTX
SUF_MAIN = 56
SUF_TAIL = 8
SUF_PAD = SUF_MAIN + SUF_TAIL
B = 1024
SEQ_PAD = 128

NC, NS = 2, 16
NW = NC * NS
NCHUNK = 4
BC = B // NCHUNK
BPW = BC // NW
CTX_CH = 2
TAIL_CH = 4


def _sc_body(cls_w, cls_p, cls_c, cls_t,
             pre_n, ctx_n, suf_n, pre_p, ctx_p, suf_p,
             tail_n, tail_p, tok_n, tok_p,
             out_pre, out_ctx, out_suf, out_tok,
             idx_v, idxp_v, idxc_v, idxt_v,
             pbuf, tbuf, cbuf0, cbuf1, sbuf0, sbuf1, lbuf,
             gsem_p, ssem_p, gsem_t, ssem_t, gsem_l, ssem_l,
             gsem_c0, gsem_c1, ssem_c0, ssem_c1,
             gsem_s0, gsem_s1, ssem_s0, ssem_s1):
    wid = lax.axis_index("s") * NC + lax.axis_index("c")
    base = wid * BPW

    pltpu.sync_copy(cls_w.at[wid], idx_v)
    pltpu.sync_copy(cls_p.at[wid], idxp_v)
    pltpu.sync_copy(cls_c.at[wid], idxc_v)
    pltpu.sync_copy(cls_t.at[wid], idxt_v)

    slots = {
        "s0": (sbuf0, gsem_s0, ssem_s0),
        "s1": (sbuf1, gsem_s1, ssem_s1),
        "c0": (cbuf0, gsem_c0, ssem_c0),
        "c1": (cbuf1, gsem_c1, ssem_c1),
        "p": (pbuf, gsem_p, ssem_p),
        "t": (tbuf, gsem_t, ssem_t),
        "l": (lbuf, gsem_l, ssem_l),
    }

    task_list = []
    halves = ((pre_n, ctx_n, suf_n, tail_n, tok_n, 0),
              (pre_p, ctx_p, suf_p, tail_p, tok_p, 1))
    for pol, (pre_t, ctx_t, suf_t, tail_t, tok_t, _) in enumerate(halves):
        ob = pol * BC + base
        for j in range(BPW):
            idx = idxc_v.at[pol * (BPW // CTX_CH) + j // CTX_CH,
                            pl.ds(j % CTX_CH, 1)]
            dst = out_suf.at[pl.ds(ob + j, 1), pl.ds(0, SUF_MAIN), :]
            task_list.append((suf_t.at[idx, pl.ds(0, SUF_MAIN), :], dst,
                              "s0" if j % 2 == 0 else "s1"))
            if j % CTX_CH == 0:
                jc = j // CTX_CH
                idxc = idxc_v.at[pol * (BPW // CTX_CH) + jc]
                cdst = out_ctx.at[pl.ds(ob + jc * CTX_CH, CTX_CH)]
                task_list.append((ctx_t.at[idxc], cdst,
                                  "c0" if jc % 2 == 0 else "c1"))
            if j % TAIL_CH == 0:
                jt = j // TAIL_CH
                idxt = idxt_v.at[pol * (BPW // TAIL_CH) + jt]
                ldst = out_suf.at[pl.ds(ob + jt * TAIL_CH, TAIL_CH),
                                  pl.ds(SUF_MAIN, SUF_TAIL), :]
                task_list.append((tail_t.at[idxt], ldst, "l"))
            if j == 1:
                task_list.append((pre_t.at[idxp_v.at[pol]],
                                  out_pre.at[pl.ds(ob, BPW)], "p"))
            if j == 3:
                task_list.append((tok_t.at[idxp_v.at[pol]],
                                  out_tok.at[pl.ds(ob, BPW), :], "t"))

    last_scatter = {}
    prev = None
    for src, dst, slot in task_list:
        buf, gsem, _ = slots[slot]
        if slot in last_scatter:
            last_scatter.pop(slot).wait()
        g = pltpu.async_copy(src, buf, gsem)
        if prev is not None:
            pg, pdst, pslot = prev
            pbuf_, _, pssem = slots[pslot]
            pg.wait()
            last_scatter[pslot] = pltpu.async_copy(pbuf_, pdst, pssem)
        prev = (g, dst, slot)
    pg, pdst, pslot = prev
    pbuf_, _, pssem = slots[pslot]
    pg.wait()
    last_scatter[pslot] = pltpu.async_copy(pbuf_, pdst, pssem)
    for s in last_scatter.values():
        s.wait()


@functools.lru_cache(maxsize=None)
def _make_call():
    mesh = plsc.VectorSubcoreMesh(core_axis_name="c", subcore_axis_name="s",
                                  num_cores=NC, num_subcores=NS)
    return pl.kernel(
        _sc_body,
        out_type=(
            jax.ShapeDtypeStruct((2 * BC, D), jnp.float32),
            jax.ShapeDtypeStruct((2 * BC, N_CTX, D), jnp.float32),
            jax.ShapeDtypeStruct((2 * BC, SUF_PAD, D), jnp.float32),
            jax.ShapeDtypeStruct((2 * BC, SEQ_PAD), jnp.int32),
        ),
        mesh=mesh,
        scratch_types=[
            pltpu.VMEM((BPW,), jnp.int32),
            pltpu.VMEM((2, BPW), jnp.int32),
            pltpu.VMEM((2 * BPW // CTX_CH, CTX_CH), jnp.int32),
            pltpu.VMEM((2 * BPW // TAIL_CH, TAIL_CH), jnp.int32),
            pltpu.VMEM((BPW, D), jnp.float32),
            pltpu.VMEM((BPW, SEQ_PAD), jnp.int32),
            pltpu.VMEM((CTX_CH, N_CTX, D), jnp.float32),
            pltpu.VMEM((CTX_CH, N_CTX, D), jnp.float32),
            pltpu.VMEM((1, SUF_MAIN, D), jnp.float32),
            pltpu.VMEM((1, SUF_MAIN, D), jnp.float32),
            pltpu.VMEM((TAIL_CH, SUF_TAIL, D), jnp.float32),
        ] + [pltpu.SemaphoreType.DMA] * 14,
    )


def _tc_asm_body(*refs):
    pre_ref, ctx_ref, suf_ref, tokc_ref = refs[-6:-2]
    out_ref, otok_ref = refs[-2:]
    out_ref[:, 0, :] = pre_ref[...]
    out_ref[:, 1:1 + N_CTX, :] = ctx_ref[...]
    out_ref[:, 1 + N_CTX:SEQ, :] = suf_ref[:, :SUF, :]
    otok_ref[...] = tokc_ref[...]


def _tc_asm(i, prompts, tokenized, pre, ctx, suf, tok):
    blk = 32
    grid = (2, BC // blk)
    nb, cb = B // blk, BC // blk

    def seg_map(pol, s):
        return (pol * cb + s, 0)

    def seg3_map(pol, s):
        return (pol * cb + s, 0, 0)

    def out_map(pol, s):
        return (pol * nb + i * cb + s, 0, 0)

    def otok_map(pol, s):
        return (pol * nb + i * cb + s, 0)

    seg_specs = [
        pl.BlockSpec((blk, D), seg_map),
        pl.BlockSpec((blk, N_CTX, D), seg3_map),
        pl.BlockSpec((blk, SUF_PAD, D), seg3_map),
        pl.BlockSpec((blk, SEQ_PAD), seg_map),
    ]
    out_shape = (
        jax.ShapeDtypeStruct((2 * B, SEQ, D), jnp.float32),
        jax.ShapeDtypeStruct((2 * B, SEQ_PAD), jnp.int32),
    )
    out_specs = (
        pl.BlockSpec((blk, SEQ, D), out_map),
        pl.BlockSpec((blk, SEQ_PAD), otok_map),
    )
    if i == 0:
        return pl.pallas_call(
            _tc_asm_body, grid=grid, in_specs=seg_specs,
            out_specs=out_specs, out_shape=out_shape,
        )(pre, ctx, suf, tok)
    return pl.pallas_call(
        _tc_asm_body, grid=grid,
        in_specs=[pl.BlockSpec(memory_space=pl.ANY),
                  pl.BlockSpec(memory_space=pl.ANY)] + seg_specs,
        out_specs=out_specs, out_shape=out_shape,
        input_output_aliases={0: 0, 1: 1},
    )(prompts, tokenized, pre, ctx, suf, tok)


@jax.jit
def _prompt_gather(cls_id, ctx_pos, ctx_neg, pre_pos2, suf_pos,
                   pre_neg2, suf_neg, tail_pos, tail_neg, tok_neg, tok_pos):
    call = _make_call()
    prompts, tokenized = None, None
    for i in range(NCHUNK):
        cls_c = lax.dynamic_slice_in_dim(cls_id, i * BC, BC)
        cls_w = cls_c.reshape(NW, BPW)
        cls2 = jnp.concatenate([cls_w] * 2, axis=1)
        pre, ctx, suf, tok = call(
            cls_w,
            cls2.reshape(NW, 2, BPW),
            cls2.reshape(NW, 2 * BPW // CTX_CH, CTX_CH),
            cls2.reshape(NW, 2 * BPW // TAIL_CH, TAIL_CH),
            pre_neg2, ctx_neg, suf_neg,
            pre_pos2, ctx_pos, suf_pos,
            tail_neg, tail_pos, tok_neg, tok_pos)
        prompts, tokenized = _tc_asm(i, prompts, tokenized,
                                     pre, ctx, suf, tok)
    return prompts, tokenized[:, :SEQ]


def kernel(cls_id, ctx_pos, ctx_neg, token_prefix_pos, token_suffix_pos,
           token_prefix_neg, token_suffix_neg, tokenized_prompts):
    n_cls = ctx_pos.shape[0]
    pad_tail = ((0, 0), (0, SUF_TAIL - (SUF - SUF_MAIN)), (0, 0))
    return _prompt_gather(
        cls_id, ctx_pos, ctx_neg,
        token_prefix_pos.reshape(n_cls, D),
        token_suffix_pos,
        token_prefix_neg.reshape(n_cls, D),
        token_suffix_neg,
        jnp.pad(token_suffix_pos[:, SUF_MAIN:, :], pad_tail),
        jnp.pad(token_suffix_neg[:, SUF_MAIN:, :], pad_tail),
        jnp.pad(tokenized_prompts[:n_cls], ((0, 0), (0, SEQ_PAD - SEQ))),
        jnp.pad(tokenized_prompts[n_cls:], ((0, 0), (0, SEQ_PAD - SEQ))),
    )
